# baseline (speedup 1.0000x reference)
import functools

import jax
import jax.numpy as jnp
from jax.experimental import pallas as pl
from jax.experimental.pallas import tpu as pltpu


def _glu_kernel(x_ref, wgv_ref, wout_ref, o_ref, p_ref, *, rank, nk):
    k = pl.program_id(1)
    partial = jax.lax.dot_general(
        x_ref[...], wgv_ref[...],
        dimension_numbers=(((1,), (1,)), ((), ())),
        preferred_element_type=jnp.float32,
    )

    @pl.when(k == 0)
    def _init():
        p_ref[...] = partial

    @pl.when(k > 0)
    def _acc():
        p_ref[...] += partial

    @pl.when(k == nk - 1)
    def _emit():
        p = p_ref[...]
        h = jax.nn.sigmoid(p[:, :rank]) * p[:, rank:]
        o_ref[...] = jax.lax.dot_general(
            h, wout_ref[...],
            dimension_numbers=(((1,), (0,)), ((), ())),
            preferred_element_type=jnp.float32,
        )


@functools.partial(jax.jit, static_argnames=())
def kernel(x, W_in, A_log, A_imag, W_dt, W_phase, W_out):
    del A_log, A_imag, W_dt, W_phase
    m, d_model = x.shape
    rank = W_out.shape[1]
    w_gv = W_in[: 2 * rank]
    w_out_t = W_out.T

    tm = 2048
    nk = 2
    tk = d_model // nk
    grid = (m // tm, nk)
    return pl.pallas_call(
        functools.partial(_glu_kernel, rank=rank, nk=nk),
        grid=grid,
        in_specs=[
            pl.BlockSpec((tm, tk), lambda i, k: (i, k)),
            pl.BlockSpec((2 * rank, tk), lambda i, k: (0, k)),
            pl.BlockSpec((rank, d_model), lambda i, k: (0, 0)),
        ],
        out_specs=pl.BlockSpec((tm, d_model), lambda i, k: (i, 0)),
        out_shape=jax.ShapeDtypeStruct((m, d_model), jnp.float32),
        scratch_shapes=[pltpu.VMEM((tm, 2 * rank), jnp.float32)],
        compiler_params=pltpu.CompilerParams(
            dimension_semantics=("parallel", "arbitrary"),
        ),
    )(x, w_gv, w_out_t)
